# Initial kernel scaffold; baseline (speedup 1.0000x reference)
import jax
import jax.numpy as jnp
from jax.experimental import pallas as pl


def kernel(ids1, ids2, pos_table, re3_table):
    raise NotImplementedError("write your pallas kernel here")


# speedup vs baseline: 2.4693x; 2.4693x over previous
import jax
import jax.numpy as jnp
from jax import lax
from jax.experimental import pallas as pl
from jax.experimental.pallas import tpu as pltpu
from jax.experimental.pallas import tpu_sc as plsc

_TABLE_ROWS = 101027
_NC = 2
_NS = 16
_NW = _NC * _NS
_SLAB = 1024
_GCHUNK = 128
_DP = 32


def _sc_body(table_hbm, idx_hbm, out_hbm, idx_v, rows_v, sem):
    nrows = idx_hbm.shape[0]
    rpw = nrows // _NW
    wid = lax.axis_index("s") * _NC + lax.axis_index("c")
    nchunks = (2 * _SLAB) // _GCHUNK

    def slab_body(si, _):
        base = wid * rpw + si * (2 * _SLAB)
        pltpu.sync_copy(idx_hbm.at[pl.ds(base, 2 * _SLAB)], idx_v)

        def fire(j, _):
            pltpu.async_copy(
                table_hbm.at[idx_v.at[pl.ds(j * _GCHUNK, _GCHUNK)]],
                rows_v.at[pl.ds(j * _GCHUNK, _GCHUNK)],
                sem)
            return 0

        lax.fori_loop(0, nchunks, fire, 0)

        def drain(j, _):
            pltpu.make_async_copy(
                table_hbm.at[idx_v.at[pl.ds(j * _GCHUNK, _GCHUNK)]],
                rows_v.at[pl.ds(j * _GCHUNK, _GCHUNK)],
                sem).wait()
            return 0

        lax.fori_loop(0, nchunks, drain, 0)

        pltpu.sync_copy(rows_v, out_hbm.at[pl.ds(base, 2 * _SLAB)])
        return 0

    lax.fori_loop(0, rpw // (2 * _SLAB), slab_body, 0)


def kernel(ids1, ids2, pos_table, re3_table):
    B, L = ids1.shape
    D = pos_table.shape[1]
    n = B * L
    assert n % (_NW * _SLAB) == 0

    z = jnp.zeros((1, D), jnp.float32)
    table = jnp.concatenate(
        [z, pos_table, z, re3_table[:, :D],
         z, pos_table, z, re3_table[:, D:]], axis=0)
    table = jnp.pad(table, ((0, 0), (0, _DP - D)))
    idx_all = jnp.stack(
        [ids1.reshape(n).astype(jnp.int32),
         ids2.reshape(n).astype(jnp.int32) + _TABLE_ROWS],
        axis=1).reshape(2 * n)

    mesh = plsc.VectorSubcoreMesh(core_axis_name="c", subcore_axis_name="s")
    out = pl.kernel(
        _sc_body,
        out_type=jax.ShapeDtypeStruct((2 * n, _DP), jnp.float32),
        mesh=mesh,
        scratch_types=[
            pltpu.VMEM((2 * _SLAB,), jnp.int32),
            pltpu.VMEM((2 * _SLAB, _DP), jnp.float32),
            pltpu.SemaphoreType.DMA,
        ],
        compiler_params=pltpu.CompilerParams(use_tc_tiling_on_sc=False),
    )(table, idx_all)
    return out[:, :D].reshape(B, L, 2 * D)


# speedup vs baseline: 2.4718x; 1.0010x over previous
import jax
import jax.numpy as jnp
from jax import lax
from jax.experimental import pallas as pl
from jax.experimental.pallas import tpu as pltpu
from jax.experimental.pallas import tpu_sc as plsc

_TABLE_ROWS = 101027
_NC = 2
_NS = 16
_NW = _NC * _NS
_SLAB = 1024
_GCHUNK = 128
_DP = 32


def _sc_body(table_hbm, idx_hbm, out_hbm, idx_v, rows_v, sem):
    nrows = idx_hbm.shape[0]
    rpw = nrows // _NW
    wid = lax.axis_index("s") * _NC + lax.axis_index("c")
    nchunks = (2 * _SLAB) // _GCHUNK

    def slab_body(si, _):
        base = wid * rpw + si * (2 * _SLAB)
        pltpu.sync_copy(idx_hbm.at[pl.ds(base, 2 * _SLAB)], idx_v)

        def fire(j, _):
            pltpu.async_copy(
                table_hbm.at[idx_v.at[pl.ds(j * _GCHUNK, _GCHUNK)]],
                rows_v.at[pl.ds(j * _GCHUNK, _GCHUNK)],
                sem)
            return 0

        lax.fori_loop(0, nchunks, fire, 0)

        def drain(j, _):
            pltpu.make_async_copy(
                table_hbm.at[idx_v.at[pl.ds(j * _GCHUNK, _GCHUNK)]],
                rows_v.at[pl.ds(j * _GCHUNK, _GCHUNK)],
                sem).wait()
            return 0

        lax.fori_loop(0, nchunks, drain, 0)

        pltpu.sync_copy(rows_v, out_hbm.at[pl.ds(base, 2 * _SLAB)])
        return 0

    lax.fori_loop(0, rpw // (2 * _SLAB), slab_body, 0)


def kernel(ids1, ids2, pos_table, re3_table):
    B, L = ids1.shape
    D = pos_table.shape[1]
    n = B * L
    assert n % (_NW * _SLAB) == 0

    table = jnp.concatenate([
        jnp.zeros((4, _DP), jnp.float32),
        jnp.pad(pos_table, ((0, 0), (0, _DP - D))),
        jnp.pad(re3_table[:, :D], ((1, 2), (0, _DP - D))),
        jnp.pad(pos_table, ((0, 0), (0, _DP - D))),
        jnp.pad(re3_table[:, D:], ((1, 2), (0, _DP - D))),
    ], axis=0)
    idx_all = jnp.stack(
        [ids1.reshape(n).astype(jnp.int32) + 3,
         ids2.reshape(n).astype(jnp.int32) + 101031],
        axis=1).reshape(2 * n)

    mesh = plsc.VectorSubcoreMesh(core_axis_name="c", subcore_axis_name="s")
    out = pl.kernel(
        _sc_body,
        out_type=jax.ShapeDtypeStruct((2 * n, _DP), jnp.float32),
        mesh=mesh,
        scratch_types=[
            pltpu.VMEM((2 * _SLAB,), jnp.int32),
            pltpu.VMEM((2 * _SLAB, _DP), jnp.float32),
            pltpu.SemaphoreType.DMA,
        ],
        compiler_params=pltpu.CompilerParams(use_tc_tiling_on_sc=False),
    )(table, idx_all)
    return out[:, :D].reshape(B, L, 2 * D)


# speedup vs baseline: 2.4878x; 1.0065x over previous
import jax
import jax.numpy as jnp
from jax import lax
from jax.experimental import pallas as pl
from jax.experimental.pallas import tpu as pltpu
from jax.experimental.pallas import tpu_sc as plsc

_TABLE_ROWS = 101027
_NC = 2
_NS = 16
_NW = _NC * _NS
_SLAB = 1024
_GCHUNK = 128
_DP = 32


def _sc_body(table_hbm, idx_hbm, out_hbm, idx_v, rows_v, sem):
    nrows = idx_hbm.shape[0]
    rpw = nrows // _NW
    wid = lax.axis_index("s") * _NC + lax.axis_index("c")
    nchunks = (2 * _SLAB) // _GCHUNK

    def slab_body(si, _):
        base = wid * rpw + si * (2 * _SLAB)
        pltpu.sync_copy(idx_hbm.at[pl.ds(base, 2 * _SLAB)], idx_v)

        def fire(j, _):
            pltpu.async_copy(
                table_hbm.at[idx_v.at[pl.ds(j * _GCHUNK, _GCHUNK)]],
                rows_v.at[pl.ds(j * _GCHUNK, _GCHUNK)],
                sem)
            return 0

        lax.fori_loop(0, nchunks, fire, 0)

        def drain(j, _):
            pltpu.make_async_copy(
                table_hbm.at[idx_v.at[pl.ds(j * _GCHUNK, _GCHUNK)]],
                rows_v.at[pl.ds(j * _GCHUNK, _GCHUNK)],
                sem).wait()
            return 0

        lax.fori_loop(0, nchunks, drain, 0)

        pltpu.sync_copy(rows_v, out_hbm.at[pl.ds(base, 2 * _SLAB)])
        return 0

    lax.fori_loop(0, rpw // (2 * _SLAB), slab_body, 0)


def kernel(ids1, ids2, pos_table, re3_table):
    B, L = ids1.shape
    D = pos_table.shape[1]
    n = B * L
    assert n % (_NW * _SLAB) == 0

    table = jnp.concatenate([
        jnp.zeros((4, 4 * _DP), jnp.float32),
        jnp.pad(pos_table, ((0, 0), (0, 4 * _DP - D))),
        jnp.pad(re3_table[:, :D], ((1, 2), (0, 4 * _DP - D))),
        jnp.pad(pos_table, ((0, 0), (0, 4 * _DP - D))),
        jnp.pad(re3_table[:, D:], ((1, 2), (0, 4 * _DP - D))),
    ], axis=0)
    table = table.reshape(4 * 202060, _DP)
    idx_all = jnp.stack(
        [4 * ids1.reshape(n).astype(jnp.int32) + 12,
         4 * ids2.reshape(n).astype(jnp.int32) + 404124],
        axis=1).reshape(2 * n)

    mesh = plsc.VectorSubcoreMesh(core_axis_name="c", subcore_axis_name="s")
    out = pl.kernel(
        _sc_body,
        out_type=jax.ShapeDtypeStruct((2 * n, _DP), jnp.float32),
        mesh=mesh,
        scratch_types=[
            pltpu.VMEM((2 * _SLAB,), jnp.int32),
            pltpu.VMEM((2 * _SLAB, _DP), jnp.float32),
            pltpu.SemaphoreType.DMA,
        ],
        compiler_params=pltpu.CompilerParams(use_tc_tiling_on_sc=False),
    )(table, idx_all)
    return out[:, :D].reshape(B, L, 2 * D)


# speedup vs baseline: 3.5349x; 1.4209x over previous
import jax
import jax.numpy as jnp
from jax import lax
from jax.experimental import pallas as pl
from jax.experimental.pallas import tpu as pltpu
from jax.experimental.pallas import tpu_sc as plsc

_TABLE_ROWS = 101027
_NC = 2
_NS = 16
_NW = _NC * _NS
_SLAB = 1024
_GCHUNK = 128
_DP = 32


def _sc_body(table_hbm, idx_hbm, out_hbm, idx_v, rows_v, sem):
    nrows = idx_hbm.shape[0]
    rpw = nrows // _NW
    wid = lax.axis_index("s") * _NC + lax.axis_index("c")
    nchunks = (2 * _SLAB) // _GCHUNK

    def slab_body(si, _):
        base = wid * rpw + si * (2 * _SLAB)
        pltpu.sync_copy(idx_hbm.at[pl.ds(base, 2 * _SLAB)], idx_v)

        def fire(j, _):
            pltpu.async_copy(
                table_hbm.at[idx_v.at[pl.ds(j * _GCHUNK, _GCHUNK)]],
                rows_v.at[pl.ds(j * _GCHUNK, _GCHUNK)],
                sem)
            return 0

        lax.fori_loop(0, nchunks, fire, 0)

        def drain(j, _):
            pltpu.make_async_copy(
                table_hbm.at[idx_v.at[pl.ds(j * _GCHUNK, _GCHUNK)]],
                rows_v.at[pl.ds(j * _GCHUNK, _GCHUNK)],
                sem).wait()
            return 0

        lax.fori_loop(0, nchunks, drain, 0)

        pltpu.sync_copy(rows_v, out_hbm.at[pl.ds(base, 2 * _SLAB)])
        return 0

    lax.fori_loop(0, rpw // (2 * _SLAB), slab_body, 0)


def kernel(ids1, ids2, pos_table, re3_table):
    B, L = ids1.shape
    D = pos_table.shape[1]
    n = B * L
    assert n % (_NW * _SLAB) == 0

    table = jnp.concatenate([
        jnp.zeros((4, 4 * _DP), jnp.float32),
        jnp.pad(pos_table, ((0, 0), (0, 4 * _DP - D))),
        jnp.pad(re3_table[:, :D], ((1, 2), (0, 4 * _DP - D))),
        jnp.pad(pos_table, ((0, 0), (0, 4 * _DP - D))),
        jnp.pad(re3_table[:, D:], ((1, 2), (0, 4 * _DP - D))),
    ], axis=0)
    table = table.reshape(4 * 202060, _DP)
    a = 4 * ids1.reshape(n).astype(jnp.int32) + 12
    b = 4 * ids2.reshape(n).astype(jnp.int32) + 404124
    idx_all = (lax.pad(a, jnp.int32(0), [(0, 1, 1)])
               + lax.pad(b, jnp.int32(0), [(1, 0, 1)]))

    mesh = plsc.VectorSubcoreMesh(core_axis_name="c", subcore_axis_name="s")
    out = pl.kernel(
        _sc_body,
        out_type=jax.ShapeDtypeStruct((2 * n, _DP), jnp.float32),
        mesh=mesh,
        scratch_types=[
            pltpu.VMEM((2 * _SLAB,), jnp.int32),
            pltpu.VMEM((2 * _SLAB, _DP), jnp.float32),
            pltpu.SemaphoreType.DMA,
        ],
        compiler_params=pltpu.CompilerParams(use_tc_tiling_on_sc=False),
    )(table, idx_all)
    return out[:, :D].reshape(B, L, 2 * D)


# speedup vs baseline: 3.5380x; 1.0009x over previous
import jax
import jax.numpy as jnp
from jax import lax
from jax.experimental import pallas as pl
from jax.experimental.pallas import tpu as pltpu
from jax.experimental.pallas import tpu_sc as plsc

_TABLE_ROWS = 101027
_NC = 2
_NS = 16
_NW = _NC * _NS
_SLAB = 1024
_GCHUNK = 128
_DP = 32


def _sc_body(table_hbm, idx_hbm, out_hbm, idx_v, rows_v, sem):
    nrows = idx_hbm.shape[0]
    rpw = nrows // _NW
    wid = lax.axis_index("s") * _NC + lax.axis_index("c")
    nchunks = (2 * _SLAB) // _GCHUNK

    def slab_body(si, _):
        base = wid * rpw + si * (2 * _SLAB)
        pltpu.sync_copy(idx_hbm.at[pl.ds(base, 2 * _SLAB)], idx_v)

        def fire(j, _):
            pltpu.async_copy(
                table_hbm.at[idx_v.at[pl.ds(j * _GCHUNK, _GCHUNK)]],
                rows_v.at[pl.ds(j * _GCHUNK, _GCHUNK)],
                sem)
            return 0

        lax.fori_loop(0, nchunks, fire, 0)

        def drain(j, _):
            pltpu.make_async_copy(
                table_hbm.at[idx_v.at[pl.ds(j * _GCHUNK, _GCHUNK)]],
                rows_v.at[pl.ds(j * _GCHUNK, _GCHUNK)],
                sem).wait()
            return 0

        lax.fori_loop(0, nchunks, drain, 0)

        pltpu.sync_copy(rows_v, out_hbm.at[pl.ds(base, 2 * _SLAB)])
        return 0

    lax.fori_loop(0, rpw // (2 * _SLAB), slab_body, 0)


def kernel(ids1, ids2, pos_table, re3_table):
    B, L = ids1.shape
    D = pos_table.shape[1]
    n = B * L
    assert n % (_NW * _SLAB) == 0

    table = jnp.concatenate([
        jnp.zeros((4, 4 * _DP), jnp.float32),
        jnp.pad(pos_table, ((0, 0), (0, 4 * _DP - D))),
        jnp.pad(re3_table[:, :D], ((1, 2), (0, 4 * _DP - D))),
        jnp.pad(pos_table, ((0, 0), (0, 4 * _DP - D))),
        jnp.pad(re3_table[:, D:], ((1, 2), (0, 4 * _DP - D))),
    ], axis=0)
    table = table.reshape(4 * 202060, _DP)
    a = 4 * ids1.reshape(n).astype(jnp.int32) + 12
    b = 4 * ids2.reshape(n).astype(jnp.int32) + 404124
    idx_all = (lax.pad(a, jnp.int32(0), [(0, 1, 1)])
               + lax.pad(b, jnp.int32(0), [(1, 0, 1)]))

    mesh = plsc.VectorSubcoreMesh(core_axis_name="c", subcore_axis_name="s")
    out = pl.kernel(
        _sc_body,
        out_type=jax.ShapeDtypeStruct((2 * n, _DP), jnp.float32),
        mesh=mesh,
        scratch_types=[
            pltpu.VMEM((2 * _SLAB,), jnp.int32),
            pltpu.VMEM((2 * _SLAB, _DP), jnp.float32),
            pltpu.SemaphoreType.DMA,
        ],
        compiler_params=pltpu.CompilerParams(use_tc_tiling_on_sc=False),
    )(table, idx_all)
    one = (ids1[0, 0] * 0 + 1).astype(jnp.float32)
    return (out[:, :D] * one).reshape(B, L, 2 * D)


# speedup vs baseline: 5.4483x; 1.5400x over previous
import jax
import jax.numpy as jnp
from jax import lax
from jax.experimental import pallas as pl
from jax.experimental.pallas import tpu as pltpu
from jax.experimental.pallas import tpu_sc as plsc

_NC = 2
_NS = 16
_NW = _NC * _NS
_SLAB = 512
_GCHUNK = 128
_DP = 64
_TROWS = 202060


def _sc_body(table_hbm, idx_hbm, out_hbm, idx_v, rows_v, sem):
    nrows = idx_hbm.shape[0]
    rpw = nrows // _NW
    wid = lax.axis_index("s") * _NC + lax.axis_index("c")
    nchunks = (2 * _SLAB) // _GCHUNK

    def slab_body(si, _):
        base = wid * rpw + si * (2 * _SLAB)
        pltpu.sync_copy(idx_hbm.at[pl.ds(base, 2 * _SLAB)], idx_v)

        def fire(j, _):
            pltpu.async_copy(
                table_hbm.at[idx_v.at[pl.ds(j * _GCHUNK, _GCHUNK)]],
                rows_v.at[pl.ds(j * _GCHUNK, _GCHUNK)],
                sem)
            return 0

        lax.fori_loop(0, nchunks, fire, 0)

        def drain(j, _):
            pltpu.make_async_copy(
                table_hbm.at[idx_v.at[pl.ds(j * _GCHUNK, _GCHUNK)]],
                rows_v.at[pl.ds(j * _GCHUNK, _GCHUNK)],
                sem).wait()
            return 0

        lax.fori_loop(0, nchunks, drain, 0)

        pltpu.sync_copy(rows_v, out_hbm.at[pl.ds(base, 2 * _SLAB)])
        return 0

    lax.fori_loop(0, rpw // (2 * _SLAB), slab_body, 0)


def _compact_body(x_ref, o_ref):
    x = x_ref[...]
    o_ref[...] = jnp.concatenate([x[:, :25], x[:, 64:89]], axis=-1)


def kernel(ids1, ids2, pos_table, re3_table):
    B, L = ids1.shape
    D = pos_table.shape[1]
    n = B * L
    assert n % (_NW * _SLAB) == 0

    table = jnp.concatenate([
        jnp.zeros((4, 2 * _DP), jnp.float32),
        jnp.pad(pos_table, ((0, 0), (0, 2 * _DP - D))),
        jnp.pad(re3_table[:, :D], ((1, 2), (0, 2 * _DP - D))),
        jnp.pad(pos_table, ((0, 0), (0, 2 * _DP - D))),
        jnp.pad(re3_table[:, D:], ((1, 2), (0, 2 * _DP - D))),
    ], axis=0)
    table = table.reshape(2 * _TROWS, _DP)
    a = 2 * ids1.reshape(n).astype(jnp.int32) + 6
    b = 2 * ids2.reshape(n).astype(jnp.int32) + 202062
    idx_all = (lax.pad(a, jnp.int32(0), [(0, 1, 1)])
               + lax.pad(b, jnp.int32(0), [(1, 0, 1)]))

    mesh = plsc.VectorSubcoreMesh(core_axis_name="c", subcore_axis_name="s")
    pairs = pl.kernel(
        _sc_body,
        out_type=jax.ShapeDtypeStruct((2 * n, _DP), jnp.float32),
        mesh=mesh,
        scratch_types=[
            pltpu.VMEM((2 * _SLAB,), jnp.int32),
            pltpu.VMEM((2 * _SLAB, _DP), jnp.float32),
            pltpu.SemaphoreType.DMA,
        ],
        compiler_params=pltpu.CompilerParams(use_tc_tiling_on_sc=False),
    )(table, idx_all)

    pairs = pairs.reshape(n, 2 * _DP)

    blk = 3200
    out = pl.pallas_call(
        _compact_body,
        out_shape=jax.ShapeDtypeStruct((n, 2 * D), jnp.float32),
        grid=(n // blk,),
        in_specs=[pl.BlockSpec((blk, 2 * _DP), lambda i: (i, 0))],
        out_specs=pl.BlockSpec((blk, 2 * D), lambda i: (i, 0)),
    )(pairs)
    return out.reshape(B, L, 2 * D)
